# baseline (speedup 1.0000x reference)
import functools

import jax
import jax.numpy as jnp
from jax.experimental import pallas as pl
from jax.experimental.pallas import tpu as pltpu

B, H, W, C = 4, 512, 512, 3
R = 512


def _body(z_hbm, msk_ref, cond_ref, out_ref, zbuf, sems):
    i = pl.program_id(0)
    slot = jax.lax.rem(i, 2)
    nxt = jax.lax.rem(i + 1, 2)

    @pl.when(i == 0)
    def _first():
        pltpu.make_async_copy(
            z_hbm.at[i, 2], zbuf.at[slot], sems.at[slot]).start()

    @pl.when(i + 1 < B)
    def _prefetch():
        pltpu.make_async_copy(
            z_hbm.at[i + 1, 2], zbuf.at[nxt], sems.at[nxt]).start()

    pltpu.make_async_copy(
        z_hbm.at[i, 2], zbuf.at[slot], sems.at[slot]).wait()

    z = zbuf[slot]
    mk = msk_ref[0] != 0
    cnd = cond_ref[0]
    sg = 1.0 / (1.0 + jnp.exp(-cnd))
    s_a = sg[0:C, :][:, None, :]
    s_b = sg[C:2 * C, :][:, None, :]
    m3 = (z > -1.0)[None, :, :]
    mk3 = mk[None, :, :]
    val = jnp.where(m3, s_a, s_b)
    val = jnp.where(mk3, val, 0.0)
    out_ref[0] = val


@jax.jit
def _run(texc_p, msk, conde):
    return pl.pallas_call(
        _body,
        grid=(B,),
        in_specs=[
            pl.BlockSpec(memory_space=pl.ANY),
            pl.BlockSpec((1, R, W), lambda i: (i, 0, 0)),
            pl.BlockSpec((1, 2 * C, W), lambda i: (i, 0, 0)),
        ],
        out_specs=pl.BlockSpec((1, C, R, W), lambda i: (i, 0, 0, 0)),
        out_shape=jax.ShapeDtypeStruct((B, C, H, W), jnp.float32),
        scratch_shapes=[
            pltpu.VMEM((2, R, W), jnp.float32),
            pltpu.SemaphoreType.DMA((2,)),
        ],
        compiler_params=pltpu.CompilerParams(
            dimension_semantics=("arbitrary",)),
    )(texc_p, msk, conde)


def kernel(texc, view_dir, cond, msk):
    del view_dir
    texc_p = jnp.transpose(texc, (0, 3, 1, 2))
    texc_p = pltpu.with_memory_space_constraint(
        texc_p, pltpu.MemorySpace.HBM)
    conde = jnp.broadcast_to(cond[:, :, None], (B, 2 * C, W))
    out_p = _run(texc_p, msk.astype(jnp.int8), conde)
    return jnp.transpose(out_p, (0, 2, 3, 1))
